# Initial kernel scaffold; baseline (speedup 1.0000x reference)
import jax
import jax.numpy as jnp
from jax.experimental import pallas as pl


def kernel(x_in, hyperedge_index, theta_ie, prev_spk, W_in, W_e2n, w_hat, neuron_mask, group_ids, g_slow_param):
    raise NotImplementedError("write your pallas kernel here")


# speedup vs baseline: 11.2849x; 11.2849x over previous
import functools

import jax
import jax.numpy as jnp
from jax import lax
from jax.experimental import pallas as pl
from jax.experimental.pallas import tpu as pltpu
from jax.experimental.pallas import tpu_sc as plsc

N = 10000
E = 160000
C = 128
G = 64
BT = 16

NC = 2
NS = 16
NW = NC * NS
EP = 163840
EPW = EP // NW
NP = 10240
RPT = NP // NS

MBLK = 400


def _h_body(x_ref, w_ref, out_ref):
    out_ref[...] = lax.dot_general(
        w_ref[...], x_ref[...], (((0,), (1,)), ((), ())),
        preferred_element_type=jnp.float32)


def _compute_hT(x_in, W_in):
    return pl.pallas_call(
        _h_body,
        out_shape=jax.ShapeDtypeStruct((N, BT), jnp.float32),
    )(x_in, W_in)


NH = 2
EH = EPW // NH


def _sc_body(hT_hbm, src_hbm, dst_hbm, w_hbm, out_hbm,
             srcv0, srcv1, dstv0, dstv1, rows, wv, zbuf, acc, hts,
             semg0, semg1, sems):
    cidx = lax.axis_index("c")
    sidx = lax.axis_index("s")
    wid = sidx * NC + cidx
    srcvs = (srcv0, srcv1)
    dstvs = (dstv0, dstv1)
    gsems = (semg0, semg1)

    for h in range(NH):
        pltpu.async_copy(src_hbm.at[wid, h], srcvs[h], semg0)
        pltpu.async_copy(dst_hbm.at[wid, h], dstvs[h], semg1)
    pltpu.async_copy(w_hbm.at[wid], wv, sems)

    def _z(i, carry):
        zbuf[i] = jnp.zeros((16,), jnp.float32)
        return carry
    lax.fori_loop(0, RPT, _z, 0)
    pltpu.sync_copy(zbuf, acc.at[pl.ds(sidx * RPT, RPT)])
    pltpu.sync_copy(hT_hbm.at[pl.ds(sidx * (N // NS), N // NS)],
                    hts.at[pl.ds(sidx * (N // NS), N // NS)])

    for h in range(NH):
        pltpu.make_async_copy(src_hbm.at[wid, h], srcvs[h], semg0).wait()
        pltpu.make_async_copy(dst_hbm.at[wid, h], dstvs[h], semg1).wait()
    pltpu.make_async_copy(w_hbm.at[wid], wv, sems).wait()
    plsc.subcore_barrier()

    dnums = lax.GatherDimensionNumbers(
        offset_dims=(), collapsed_slice_dims=(0,), start_index_map=(0,))

    for h in range(NH):
        pltpu.async_copy(hts.at[srcvs[h]],
                         rows.at[pl.ds(h * EH, EH)], gsems[h])

    for h in range(NH):
        pltpu.make_async_copy(hts.at[srcvs[h]],
                              rows.at[pl.ds(h * EH, EH)], gsems[h]).wait()

        def _mul(g, carry2):
            base = h * EH + g * 16
            w16 = wv[pl.ds(base, 16)]
            for j in range(16):
                wj = lax.gather(
                    w16, jnp.full((16, 1), j, jnp.int32), dnums,
                    slice_sizes=(1,),
                    mode=lax.GatherScatterMode.PROMISE_IN_BOUNDS)
                rows[base + j] = rows[base + j] * wj
            return carry2
        lax.fori_loop(0, EH // 16, _mul, 0)
        pltpu.async_copy(rows.at[pl.ds(h * EH, EH)],
                         acc.at[dstvs[h]], sems, add=True)

    for h in range(NH):
        pltpu.make_async_copy(rows.at[pl.ds(h * EH, EH)],
                              acc.at[dstvs[h]], sems).wait()

    plsc.subcore_barrier()
    pltpu.sync_copy(acc.at[pl.ds(sidx * RPT, RPT)],
                    out_hbm.at[cidx, pl.ds(sidx * RPT, RPT)])


@functools.cache
def _make_sc_scatter():
    return functools.partial(
        pl.kernel,
        out_type=jax.ShapeDtypeStruct((NC, NP, BT), jnp.float32),
        mesh=plsc.VectorSubcoreMesh(core_axis_name="c", subcore_axis_name="s",
                                    num_cores=NC, num_subcores=NS),
        scratch_types=[
            pltpu.VMEM((EH,), jnp.int32),
            pltpu.VMEM((EH,), jnp.int32),
            pltpu.VMEM((EH,), jnp.int32),
            pltpu.VMEM((EH,), jnp.int32),
            pltpu.VMEM((EPW, BT), jnp.float32),
            pltpu.VMEM((EPW,), jnp.float32),
            pltpu.VMEM((RPT, BT), jnp.float32),
            pltpu.VMEM_SHARED((NP, BT), jnp.float32),
            pltpu.VMEM_SHARED((N, BT), jnp.float32),
            pltpu.SemaphoreType.DMA,
            pltpu.SemaphoreType.DMA,
            pltpu.SemaphoreType.DMA,
        ],
        compiler_params=pltpu.CompilerParams(use_tc_tiling_on_sc=False),
    )(_sc_body)


def _b_body(w_ref, acc_ref, prevc_f, prevc_b, thetac, maskc,
            gidr, gidc, prevr, x_ref, gsp_ref, out_ref, st_ref, gm_ref):
    i = pl.program_id(0)

    @pl.when(i == 0)
    def _():
        g_slow = jax.nn.sigmoid(gsp_ref[0, 0])
        gate = g_slow * (1.0 + 0.1 * prevc_f[...])
        st_ref[...] = (acc_ref[0] + acc_ref[1]) * gate
        oh = (lax.broadcasted_iota(jnp.int32, (G, N), 0)
              == gidr[...]).astype(jnp.float32)
        gsum = lax.dot_general(oh, prevr[...],
                               (((1,), (1,)), ((), ())),
                               preferred_element_type=jnp.float32)
        cnts = jnp.sum(oh, axis=1, keepdims=True)
        gm_ref[:, 0:1] = gsum / jnp.maximum(cnts, 1.0)

    mm = lax.dot_general(w_ref[...], st_ref[...],
                         (((1,), (0,)), ((), ())),
                         preferred_element_type=jnp.float32)

    ones_col = jnp.ones((BT, 1), jnp.float32)
    pre = lax.dot_general(x_ref[...], ones_col,
                          (((0,), (0,)), ((), ())),
                          preferred_element_type=jnp.float32) * (1.0 / BT)
    flag = jnp.where(i == 0, 1.0, 0.0)
    prepad = jnp.concatenate(
        [pre, jnp.zeros((MBLK - C, 1), jnp.float32)], axis=0) * flag
    prev_b = prevc_b[...]
    stdp = 0.01 * prepad * prev_b - 0.005 * prev_b

    ohb = (gidc[...] == lax.broadcasted_iota(jnp.int32, (MBLK, G), 1)
           ).astype(jnp.float32)
    inh = 0.5 * lax.dot_general(ohb, gm_ref[:, 0:1],
                                (((1,), (0,)), ((), ())),
                                preferred_element_type=jnp.float32)

    isyn = (mm + 0.1 * stdp) * maskc[...]
    mem = isyn - inh - thetac[...] - 0.5
    out_ref[...] = jax.nn.sigmoid(4.0 * mem)


def _compute_out(W_e2n, acc2, prev_col, prev_colb, theta_col, mask_col,
                 gid_row, gid_col, prev_row, x_in, gsp):
    return pl.pallas_call(
        _b_body,
        grid=(N // MBLK,),
        in_specs=[
            pl.BlockSpec((MBLK, N), lambda i: (i, 0)),
            pl.BlockSpec((NC, N, BT), lambda i: (0, 0, 0)),
            pl.BlockSpec((N, 1), lambda i: (0, 0)),
            pl.BlockSpec((MBLK, 1), lambda i: (i, 0)),
            pl.BlockSpec((MBLK, 1), lambda i: (i, 0)),
            pl.BlockSpec((MBLK, 1), lambda i: (i, 0)),
            pl.BlockSpec((1, N), lambda i: (0, 0)),
            pl.BlockSpec((MBLK, 1), lambda i: (i, 0)),
            pl.BlockSpec((1, N), lambda i: (0, 0)),
            pl.BlockSpec((BT, C), lambda i: (0, 0)),
            pl.BlockSpec((1, 1), lambda i: (0, 0)),
        ],
        out_specs=pl.BlockSpec((MBLK, BT), lambda i: (i, 0)),
        out_shape=jax.ShapeDtypeStruct((N, BT), jnp.float32),
        scratch_shapes=[
            pltpu.VMEM((N, BT), jnp.float32),
            pltpu.VMEM((G, 128), jnp.float32),
        ],
    )(W_e2n, acc2, prev_col, prev_colb, theta_col, mask_col,
      gid_row, gid_col, prev_row, x_in, gsp)


def kernel(x_in, hyperedge_index, theta_ie, prev_spk, W_in, W_e2n,
           w_hat, neuron_mask, group_ids, g_slow_param):
    hT = _compute_hT(x_in, W_in)

    pad = EP - E
    src_p = jnp.concatenate([hyperedge_index[0],
                             jnp.zeros((pad,), jnp.int32)])
    dst_p = jnp.concatenate([hyperedge_index[1],
                             jnp.arange(pad, dtype=jnp.int32) % N])
    w_p = jnp.concatenate([w_hat, jnp.zeros((pad,), jnp.float32)])
    src3 = src_p.reshape(NW, NH, EH)
    dst3 = dst_p.reshape(NW, NH, EH)
    w3 = w_p.reshape(NW, EPW)

    acc2 = _make_sc_scatter()(hT, src3, dst3, w3)

    out_T = _compute_out(
        W_e2n, acc2,
        prev_spk.reshape(N, 1), prev_spk.reshape(N, 1), theta_ie.reshape(N, 1),
        neuron_mask.reshape(N, 1), group_ids.reshape(1, N),
        group_ids.reshape(N, 1), prev_spk.reshape(1, N),
        x_in, g_slow_param.reshape(1, 1))
    return out_T.T
